# Initial kernel scaffold; baseline (speedup 1.0000x reference)
import jax
import jax.numpy as jnp
from jax.experimental import pallas as pl


def kernel(x, table):
    raise NotImplementedError("write your pallas kernel here")


# speedup vs baseline: 2.0684x; 2.0684x over previous
import functools

import jax
import jax.numpy as jnp
import numpy as np
from jax import lax
from jax.experimental import pallas as pl
from jax.experimental.pallas import tpu as pltpu
from jax.experimental.pallas import tpu_sc as plsc

_NC = 2
_NS = 16
_L = 16
_NW = _NC * _NS

_CHUNK = 64


def _sinusoid_pe_np(max_len: int, d_model: int) -> np.ndarray:
    pos = np.arange(max_len, dtype=np.float32)[:, None]
    i = np.arange(0, d_model, 2, dtype=np.float32)
    div = np.power(10000.0, i / np.float32(d_model), dtype=np.float32)
    pe = np.zeros((max_len, d_model), dtype=np.float32)
    pe[:, 0::2] = np.sin(pos / div)
    pe[:, 1::2] = np.cos(pos / div)
    return pe


@functools.partial(jax.jit, static_argnames=("seq_len",))
def _emb_call(x_flat, table, pe, seq_len):
    n = x_flat.shape[0]
    d = table.shape[1]
    per_w = n // _NW
    nchunks = per_w // _CHUNK
    mesh = plsc.VectorSubcoreMesh(core_axis_name="c", subcore_axis_name="s")

    @functools.partial(
        pl.kernel,
        out_type=jax.ShapeDtypeStruct((n, d), jnp.float32),
        mesh=mesh,
        scratch_types=[
            pltpu.VMEM((_CHUNK,), jnp.int32),
            pltpu.VMEM((_CHUNK, d), jnp.float32),
            pltpu.VMEM((_CHUNK, d), jnp.float32),
            pltpu.SemaphoreType.DMA,
        ],
    )
    def emb(idx_hbm, table_hbm, pe_hbm, out_hbm, idx_v, rows_v, pe_v, sem):
        wid = lax.axis_index("s") * _NC + lax.axis_index("c")
        base = wid * per_w
        s_base = lax.rem(base, seq_len)

        def chunk_body(c, carry):
            off = base + c * _CHUNK
            s_off = s_base + c * _CHUNK
            pltpu.sync_copy(idx_hbm.at[pl.ds(off, _CHUNK)], idx_v)
            pltpu.async_copy(table_hbm.at[idx_v], rows_v, sem).wait()
            pltpu.sync_copy(pe_hbm.at[pl.ds(s_off, _CHUNK)], pe_v)

            def row_body(r, rcarry):
                for k in range(d // _L):
                    sl = pl.ds(k * _L, _L)
                    rows_v[r, sl] = rows_v[r, sl] + pe_v[r, sl]
                return rcarry

            lax.fori_loop(0, _CHUNK, row_body, 0, unroll=False)
            pltpu.sync_copy(rows_v, out_hbm.at[pl.ds(off, _CHUNK)])
            return carry

        lax.fori_loop(0, nchunks, chunk_body, 0, unroll=False)

    return emb(x_flat, table, pe)


def kernel(x, table):
    b, s = x.shape
    d = table.shape[1]
    pe = jnp.asarray(_sinusoid_pe_np(s, d))
    x_flat = x.reshape(-1).astype(jnp.int32)
    out = _emb_call(x_flat, table, pe, s)
    return out.reshape(b, s, d)


# speedup vs baseline: 2.1513x; 1.0401x over previous
import functools

import jax
import jax.numpy as jnp
import numpy as np
from jax import lax
from jax.experimental import pallas as pl
from jax.experimental.pallas import tpu as pltpu
from jax.experimental.pallas import tpu_sc as plsc

_NC = 2
_NS = 16
_L = 16
_NW = _NC * _NS

_CS = 16


def _sinusoid_pe_np(max_len: int, d_model: int) -> np.ndarray:
    pos = np.arange(max_len, dtype=np.float32)[:, None]
    i = np.arange(0, d_model, 2, dtype=np.float32)
    div = np.power(10000.0, i / np.float32(d_model), dtype=np.float32)
    pe = np.zeros((max_len, d_model), dtype=np.float32)
    pe[:, 0::2] = np.sin(pos / div)
    pe[:, 1::2] = np.cos(pos / div)
    return pe


@functools.partial(jax.jit, static_argnames=("nb", "seq_len"))
def _emb_call(x_flat, table, pe, nb, seq_len):
    n = x_flat.shape[0]
    d = table.shape[1]
    sr = seq_len // _NW
    nch = sr // _CS
    rows_per_chunk = nb * _CS
    mesh = plsc.VectorSubcoreMesh(core_axis_name="c", subcore_axis_name="s")

    @functools.partial(
        pl.kernel,
        out_type=jax.ShapeDtypeStruct((n, d), jnp.float32),
        mesh=mesh,
        scratch_types=[
            pltpu.VMEM((nb, sr), jnp.int32),
            pltpu.VMEM((2, rows_per_chunk, d), jnp.float32),
            pltpu.VMEM((2, _CS, d), jnp.float32),
            pltpu.SemaphoreType.DMA,
            pltpu.SemaphoreType.DMA,
            pltpu.SemaphoreType.DMA,
            pltpu.SemaphoreType.DMA,
            pltpu.SemaphoreType.DMA,
            pltpu.SemaphoreType.DMA,
        ],
    )
    def emb(idx_hbm, table_hbm, pe_hbm, out_hbm,
            idx_all, rows, pe2, g0, g1, p0, p1, o0, o1):
        gsem = (g0, g1)
        psem = (p0, p1)
        osem = (o0, o1)
        wid = lax.axis_index("s") * _NC + lax.axis_index("c")
        s0 = wid * sr

        for b in range(nb):
            pltpu.sync_copy(idx_hbm.at[pl.ds(b * seq_len + s0, sr)],
                            idx_all.at[b])

        def start(c, q):
            gd = [
                pltpu.async_copy(
                    table_hbm.at[idx_all.at[b, pl.ds(c * _CS, _CS)]],
                    rows.at[q, pl.ds(b * _CS, _CS)],
                    gsem[q],
                )
                for b in range(nb)
            ]
            pd = pltpu.async_copy(pe_hbm.at[pl.ds(s0 + c * _CS, _CS)],
                                  pe2.at[q], psem[q])
            return gd, pd

        def start_out(c, q):
            return [
                pltpu.async_copy(
                    rows.at[q, pl.ds(b * _CS, _CS)],
                    out_hbm.at[pl.ds(b * seq_len + s0 + c * _CS, _CS)],
                    osem[q],
                )
                for b in range(nb)
            ]

        inflight = {0: start(0, 0)}
        out_inflight = {}
        for c in range(nch):
            q = c % 2
            if c + 1 < nch:
                if c - 1 in out_inflight:
                    for od in out_inflight.pop(c - 1):
                        od.wait()
                inflight[c + 1] = start(c + 1, 1 - q)
            gd, pd = inflight.pop(c)
            for gdesc in gd:
                gdesc.wait()
            pd.wait()

            def row_body(j, carry):
                for k in range(d // _L):
                    sl = pl.ds(k * _L, _L)
                    pvec = pe2[q, j, sl]
                    for b in range(nb):
                        r = b * _CS + j
                        rows[q, r, sl] = rows[q, r, sl] + pvec
                return carry

            lax.fori_loop(0, _CS, row_body, 0, unroll=False)
            out_inflight[c] = start_out(c, q)
        for ods in out_inflight.values():
            for od in ods:
                od.wait()

    return emb(x_flat, table, pe)


def kernel(x, table):
    b, s = x.shape
    d = table.shape[1]
    pe = jnp.asarray(_sinusoid_pe_np(s, d))
    x_flat = x.reshape(-1).astype(jnp.int32)
    out = _emb_call(x_flat, table, pe, b, s)
    return out.reshape(b, s, d)
